# baseline (speedup 1.0000x reference)
import functools

import jax
import jax.numpy as jnp
from jax import lax
from jax.experimental import pallas as pl
from jax.experimental.pallas import tpu as pltpu
from jax.experimental.pallas import tpu_sc as plsc

_E = 320000
_D = 128
_N0, _N1, _N2 = 5, 6, 2
_NT = _N0 * _N1 * _N2

_NC = 2
_NS = 16
_NW = _NC * _NS
_PER_W = _E // _NW
_SB = 80
_GPC = 5
_CH = _SB * _GPC
_NCHUNK = _PER_W // _CH

_BB = 8000


def _table_body(w0_ref, w1_ref, w2_ref, t_ref):
    r = lax.broadcasted_iota(jnp.int32, (_NT, 1), 0)
    oh0 = (lax.broadcasted_iota(jnp.int32, (_NT, _N0), 1)
           == r // (_N1 * _N2)).astype(jnp.float32)
    oh1 = (lax.broadcasted_iota(jnp.int32, (_NT, _N1), 1)
           == (r // _N2) % _N1).astype(jnp.float32)
    oh2 = (lax.broadcasted_iota(jnp.int32, (_NT, _N2), 1)
           == r % _N2).astype(jnp.float32)
    t_ref[...] = (
        jnp.dot(oh0, w0_ref[...], preferred_element_type=jnp.float32,
                  precision=lax.Precision.HIGHEST)
        + jnp.dot(oh1, w1_ref[...], preferred_element_type=jnp.float32,
                  precision=lax.Precision.HIGHEST)
        + jnp.dot(oh2, w2_ref[...], preferred_element_type=jnp.float32,
                  precision=lax.Precision.HIGHEST)
    )


_build_table = pl.pallas_call(
    _table_body,
    out_shape=jax.ShapeDtypeStruct((_NT, _D), jnp.float32),
)


def _cidx_body(ea_ref, out_ref):
    ea = ea_ref[...]
    out_ref[...] = ea[:, 0:1] * (_N1 * _N2) + ea[:, 1:2] * _N2 + ea[:, 2:3]


_build_cidx = pl.pallas_call(
    _cidx_body,
    grid=(_E // _BB,),
    in_specs=[pl.BlockSpec((_BB, 3), lambda i: (i, 0))],
    out_specs=pl.BlockSpec((_BB, 1), lambda i: (i, 0)),
    out_shape=jax.ShapeDtypeStruct((_E, 1), jnp.int32),
)


@functools.cache
def _make_sc_gather():
    @functools.partial(
        pl.kernel,
        mesh=plsc.VectorSubcoreMesh(core_axis_name="c", subcore_axis_name="s"),
        out_type=jax.ShapeDtypeStruct((_E, _D), jnp.float32),
        scratch_types=[
            pltpu.VMEM((_PER_W // _SB, _SB), jnp.int32),
            pltpu.VMEM((_CH, _D), jnp.float32),
            pltpu.SemaphoreType.DMA,
        ],
    )
    def _sc_gather(t_hbm, idx_hbm, out_hbm, idx_v, rows_v, sem):
        wid = lax.axis_index("s") * _NC + lax.axis_index("c")
        base = wid * _PER_W
        pltpu.sync_copy(idx_hbm.at[wid], idx_v)

        def chunk(c, carry):
            cps = [
                pltpu.async_copy(
                    t_hbm.at[idx_v.at[c * _GPC + g]],
                    rows_v.at[pl.ds(g * _SB, _SB)],
                    sem,
                )
                for g in range(_GPC)
            ]
            for cp in cps:
                cp.wait()
            pltpu.sync_copy(rows_v, out_hbm.at[pl.ds(base + c * _CH, _CH)])
            return carry

        lax.fori_loop(0, _NCHUNK, chunk, 0)

    return _sc_gather


def kernel(edge_attr, W0, W1, W2):
    ea = edge_attr.astype(jnp.int32)
    t = _build_table(W0, W1, W2)
    cidx = _build_cidx(ea)
    idx3 = cidx.reshape(_NW, _PER_W // _SB, _SB)
    return _make_sc_gather()(t, idx3)


# speedup vs baseline: 6.2355x; 6.2355x over previous
import functools

import jax
import jax.numpy as jnp
from jax import lax
from jax.experimental import pallas as pl
from jax.experimental.pallas import tpu as pltpu
from jax.experimental.pallas import tpu_sc as plsc

_E = 320000
_D = 128
_N0, _N1, _N2 = 5, 6, 2
_NT = _N0 * _N1 * _N2

_NC = 2
_NS = 16
_NW = _NC * _NS
_PER_W = _E // _NW
_SB = 80
_GPC = 5
_CH = _SB * _GPC
_NCHUNK = _PER_W // _CH

_BB = 8000


def _table_body(w0_ref, w1_ref, w2_ref, t_ref):
    r = lax.broadcasted_iota(jnp.int32, (_NT, 1), 0)
    oh0 = (lax.broadcasted_iota(jnp.int32, (_NT, _N0), 1)
           == r // (_N1 * _N2)).astype(jnp.float32)
    oh1 = (lax.broadcasted_iota(jnp.int32, (_NT, _N1), 1)
           == (r // _N2) % _N1).astype(jnp.float32)
    oh2 = (lax.broadcasted_iota(jnp.int32, (_NT, _N2), 1)
           == r % _N2).astype(jnp.float32)
    t_ref[...] = (
        jnp.dot(oh0, w0_ref[...], preferred_element_type=jnp.float32,
                  precision=lax.Precision.HIGHEST)
        + jnp.dot(oh1, w1_ref[...], preferred_element_type=jnp.float32,
                  precision=lax.Precision.HIGHEST)
        + jnp.dot(oh2, w2_ref[...], preferred_element_type=jnp.float32,
                  precision=lax.Precision.HIGHEST)
    )


_build_table = pl.pallas_call(
    _table_body,
    out_shape=jax.ShapeDtypeStruct((_NT, _D), jnp.float32),
)


def _cidx_body(ea_ref, out_ref):
    ea = ea_ref[...]
    out_ref[...] = ea[:, 0:1] * (_N1 * _N2) + ea[:, 1:2] * _N2 + ea[:, 2:3]


_build_cidx = pl.pallas_call(
    _cidx_body,
    grid=(_E // _BB,),
    in_specs=[pl.BlockSpec((_BB, 3), lambda i: (i, 0))],
    out_specs=pl.BlockSpec((_BB, 1), lambda i: (i, 0)),
    out_shape=jax.ShapeDtypeStruct((_E, 1), jnp.int32),
)


@functools.cache
def _make_sc_gather():
    @functools.partial(
        pl.kernel,
        mesh=plsc.VectorSubcoreMesh(core_axis_name="c", subcore_axis_name="s"),
        out_type=jax.ShapeDtypeStruct((_E, _D), jnp.float32),
        scratch_types=[
            pltpu.VMEM((_PER_W // _SB, _SB), jnp.int32),
            pltpu.VMEM((2, _CH, _D), jnp.float32),
            pltpu.VMEM_SHARED((_NT, _D), jnp.float32),
            pltpu.SemaphoreType.DMA,
            pltpu.SemaphoreType.DMA,
        ],
    )
    def _sc_gather(t_hbm, idx_hbm, out_hbm, idx_v, rows_v, t_sh, gsem, ssem):
        sid = lax.axis_index("s")
        wid = sid * _NC + lax.axis_index("c")
        base = wid * _PER_W
        @pl.when(sid == 0)
        def _():
            pltpu.sync_copy(t_hbm, t_sh)

        pltpu.sync_copy(idx_hbm.at[wid], idx_v)
        plsc.subcore_barrier()

        def gather_chunk(c, buf):
            return [
                pltpu.async_copy(
                    t_sh.at[idx_v.at[c * _GPC + g]],
                    rows_v.at[buf].at[pl.ds(g * _SB, _SB)],
                    gsem,
                )
                for g in range(_GPC)
            ]

        def store_chunk(c, buf):
            return pltpu.async_copy(
                rows_v.at[buf], out_hbm.at[pl.ds(base + c * _CH, _CH)], ssem
            )

        for cp in gather_chunk(0, 0):
            cp.wait()

        def chunk(c, carry):
            buf = lax.rem(c, 2)
            st = store_chunk(c, buf)
            nxt = gather_chunk(c + 1, 1 - buf)
            for cp in nxt:
                cp.wait()
            st.wait()
            return carry

        lax.fori_loop(0, _NCHUNK - 1, chunk, 0)
        store_chunk(_NCHUNK - 1, (_NCHUNK - 1) % 2).wait()

    return _sc_gather


def kernel(edge_attr, W0, W1, W2):
    ea = edge_attr.astype(jnp.int32)
    t = _build_table(W0, W1, W2)
    cidx = _build_cidx(ea)
    idx3 = cidx.reshape(_NW, _PER_W // _SB, _SB)
    return _make_sc_gather()(t, idx3)


# speedup vs baseline: 20.0516x; 3.2157x over previous
import functools

import jax
import jax.numpy as jnp
from jax import lax
from jax.experimental import pallas as pl
from jax.experimental.pallas import tpu as pltpu
from jax.experimental.pallas import tpu_sc as plsc

_E = 320000
_D = 128
_N0, _N1, _N2 = 5, 6, 2
_NT = _N0 * _N1 * _N2

_NC = 2
_NS = 16
_NW = _NC * _NS
_PER_W = _E // _NW
_SB = 80
_GPC = 5
_CH = _SB * _GPC
_NCHUNK = _PER_W // _CH

_BB = 8000


def _table_body(w0_ref, w1_ref, w2_ref, t_ref):
    r = lax.broadcasted_iota(jnp.int32, (_NT, 1), 0)
    oh0 = (lax.broadcasted_iota(jnp.int32, (_NT, _N0), 1)
           == r // (_N1 * _N2)).astype(jnp.float32)
    oh1 = (lax.broadcasted_iota(jnp.int32, (_NT, _N1), 1)
           == (r // _N2) % _N1).astype(jnp.float32)
    oh2 = (lax.broadcasted_iota(jnp.int32, (_NT, _N2), 1)
           == r % _N2).astype(jnp.float32)
    t_ref[...] = (
        jnp.dot(oh0, w0_ref[...], preferred_element_type=jnp.float32,
                  precision=lax.Precision.HIGHEST)
        + jnp.dot(oh1, w1_ref[...], preferred_element_type=jnp.float32,
                  precision=lax.Precision.HIGHEST)
        + jnp.dot(oh2, w2_ref[...], preferred_element_type=jnp.float32,
                  precision=lax.Precision.HIGHEST)
    )


_build_table = pl.pallas_call(
    _table_body,
    out_shape=jax.ShapeDtypeStruct((_NT, _D), jnp.float32),
)


def _cidx_body(ea_ref, out_ref):
    ea = ea_ref[...]
    out_ref[...] = ea[:, 0:1] * (_N1 * _N2) + ea[:, 1:2] * _N2 + ea[:, 2:3]


_build_cidx = pl.pallas_call(
    _cidx_body,
    grid=(_E // _BB,),
    in_specs=[pl.BlockSpec((_BB, 3), lambda i: (i, 0))],
    out_specs=pl.BlockSpec((_BB, 1), lambda i: (i, 0)),
    out_shape=jax.ShapeDtypeStruct((_E, 1), jnp.int32),
)


@functools.cache
def _make_sc_gather():
    @functools.partial(
        pl.kernel,
        mesh=plsc.VectorSubcoreMesh(core_axis_name="c", subcore_axis_name="s"),
        out_type=jax.ShapeDtypeStruct((_E, _D), jnp.float32),
        scratch_types=[
            pltpu.VMEM((_PER_W // _SB, _SB), jnp.int32),
            pltpu.VMEM((2, _CH, _D), jnp.float32),
            pltpu.VMEM_SHARED((_NT, _D), jnp.float32),
            pltpu.SemaphoreType.DMA,
            pltpu.SemaphoreType.DMA,
        ],
    )
    def _sc_gather(t_hbm, idx_hbm, out_hbm, idx_v, rows_v, t_sh, gsem, ssem):
        sid = lax.axis_index("s")
        wid = sid * _NC + lax.axis_index("c")
        base = wid * _PER_W
        @pl.when(sid == 0)
        def _():
            pltpu.sync_copy(t_hbm, t_sh)

        pltpu.sync_copy(idx_hbm.at[wid], idx_v)
        plsc.subcore_barrier()

        def gather_chunk(c, buf):
            return [
                pltpu.async_copy(
                    t_sh.at[idx_v.at[c * _GPC + g]],
                    rows_v.at[buf].at[pl.ds(g * _SB, _SB)],
                    gsem,
                )
                for g in range(_GPC)
            ]

        def store_chunk(c, buf):
            return pltpu.async_copy(
                rows_v.at[buf], out_hbm.at[pl.ds(base + c * _CH, _CH)], ssem
            )

        for cp in gather_chunk(0, 0):
            cp.wait()

        def chunk(c, carry):
            buf = lax.rem(c, 2)
            st = store_chunk(c, buf)
            nxt = gather_chunk(c + 1, 1 - buf)
            for cp in nxt:
                cp.wait()
            st.wait()
            return carry

        lax.fori_loop(0, _NCHUNK - 1, chunk, 0)
        store_chunk(_NCHUNK - 1, (_NCHUNK - 1) % 2).wait()

    return _sc_gather


def kernel(edge_attr, W0, W1, W2):
    ea = edge_attr.astype(jnp.int32)
    t = _build_table(W0, W1, W2)
    cidx = ea[:, 0] * 12 + ea[:, 1] * 2 + ea[:, 2]
    idx3 = cidx.reshape(_NW, _PER_W // _SB, _SB)
    return _make_sc_gather()(t, idx3)
